# baseline (speedup 1.0000x reference)
import jax
import jax.numpy as jnp
from jax import lax
from jax.experimental import pallas as pl
from jax.experimental.pallas import tpu as pltpu
from jax.experimental.pallas import tpu_sc as plsc

BATCH = 16384
DIM = 32
OUT_D = 2 * DIM + 1
VROWS = 1000001
NBLK = VROWS // 128
TAIL0 = NBLK * 128
NTAIL = VROWS - TAIL0
PACKW = TAIL0
NBND_PAD = 1024
LANES = 16

_NC, _NS = 2, 16
NW = _NC * _NS
B_PER_W = BATCH // NW
N_CHUNK = B_PER_W // LANES
BLK_PER_W = -(-NBLK // NW)
CHUNK = 7 * 128
N_CHUNKS_A = BLK_PER_W * 128 // CHUNK


def _repack_body(utab_t, flat_out, in_v, sem_i0, sem_i1, sem_i2,
                 sem_o0, sem_o1, sem_o2):
    wid = lax.axis_index("s") * _NC + lax.axis_index("c")
    c0w = jnp.minimum(wid * BLK_PER_W, NBLK - BLK_PER_W) * 128

    in_sems = [sem_i0, sem_i1, sem_i2]
    out_sems = [sem_o0, sem_o1, sem_o2]

    def issue_in(c0, buf):
        pltpu.async_copy(utab_t.at[pl.ds(0, DIM), pl.ds(c0, CHUNK)],
                         in_v.at[buf], in_sems[buf])

    def drain(buf, sem):
        pltpu.make_async_copy(utab_t.at[pl.ds(0, DIM), pl.ds(0, CHUNK)],
                              in_v.at[buf], sem).wait()

    issue_in(c0w, 0)

    def chunk_body(i, carry):
        c0 = c0w + i * CHUNK
        for cbuf in range(3):
            @pl.when(lax.rem(i, 3) == cbuf)
            def _():
                nbuf = (cbuf + 1) % 3

                @pl.when(i + 1 < N_CHUNKS_A)
                def _():
                    @pl.when(i >= 2)
                    def _():
                        drain(nbuf, out_sems[nbuf])
                    issue_in(c0 + CHUNK, nbuf)

                drain(cbuf, in_sems[cbuf])
                for j in range(DIM):
                    pltpu.async_copy(
                        in_v.at[cbuf, j],
                        flat_out.at[pl.ds(j * PACKW + c0, CHUNK)],
                        out_sems[cbuf])
        return carry

    lax.fori_loop(0, N_CHUNKS_A, chunk_body, 0)
    for cbuf in range(3):
        drain(cbuf, out_sems[cbuf])


def _lookup_body(uid_hbm, ts_hbm, flat_tab, ttab_hbm, bnd_hbm, norm_hbm,
                 tail_hbm, out_hbm,
                 idx_v, ts_v, bnd_v, cidx_v, cols_v, ttab_v, tail_v, norm_v,
                 sem_u):
    wid = lax.axis_index("s") * _NC + lax.axis_index("c")
    base = wid * B_PER_W

    pltpu.sync_copy(uid_hbm.at[pl.ds(base, B_PER_W)], idx_v)

    def cidx_body(j, off):
        for g in range(N_CHUNK):
            sl = pl.ds(g * LANES, LANES)
            iv = jnp.minimum(idx_v[sl], TAIL0 - 1)
            cidx_v[j, sl] = iv + off
        return off + PACKW

    lax.fori_loop(0, DIM, cidx_body, jnp.zeros((LANES,), jnp.int32))
    copies = [
        pltpu.async_copy(flat_tab.at[cidx_v.at[j]], cols_v.at[j], sem_u)
        for j in range(DIM)
    ]

    pltpu.sync_copy(ts_hbm.at[pl.ds(base, B_PER_W)], ts_v)
    pltpu.sync_copy(ttab_hbm, ttab_v)
    pltpu.sync_copy(tail_hbm, tail_v)
    pltpu.sync_copy(bnd_hbm, bnd_v)
    pltpu.sync_copy(norm_hbm, norm_v)
    mean = norm_v[pl.ds(0, LANES)]
    denom = norm_v[pl.ds(LANES, LANES)]

    def group_body(g, carry):
        sl = pl.ds(g * LANES, LANES)
        t = ts_v[sl]
        pos = jnp.zeros((LANES,), jnp.int32)
        step = NBND_PAD // 2
        while step >= 1:
            probe = pos + (step - 1)
            val = plsc.load_gather(bnd_v, [probe])
            pos = jnp.where(val <= t, pos + step, pos)
            step //= 2
        for j in range(DIM):
            tv = plsc.load_gather(ttab_v, [pos, jnp.full((LANES,), j,
                                                         jnp.int32)])
            cols_v[DIM + j, sl] = tv
        cols_v[2 * DIM, sl] = (t - mean) / denom
        return carry

    lax.fori_loop(0, N_CHUNK, group_body, 0)

    for c in copies:
        c.wait()

    def tail_body(g, carry):
        sl = pl.ds(g * LANES, LANES)
        iv = idx_v[sl]
        is_tail = iv >= TAIL0

        @pl.when(jnp.any(is_tail))
        def _():
            ti = jnp.maximum(iv - TAIL0, 0)
            for j in range(DIM):
                tv = plsc.load_gather(tail_v, [ti, jnp.full((LANES,), j,
                                                            jnp.int32)])
                cur = cols_v[j, sl]
                cols_v[j, sl] = jnp.where(is_tail, tv, cur)
        return carry

    lax.fori_loop(0, N_CHUNK, tail_body, 0)

    pltpu.sync_copy(cols_v, out_hbm.at[:, pl.ds(base, B_PER_W)])


@jax.jit
def kernel(user_id, timestamp, user_table, ts_table, boundaries, ts_mean,
           ts_var):
    idx = user_id.astype(jnp.int32)
    utab_t = user_table.T
    tail = user_table[TAIL0:]
    bnd_pad = jnp.full((NBND_PAD,), jnp.inf, jnp.float32).at[
        : boundaries.shape[0]].set(boundaries)
    norm = jnp.concatenate([
        jnp.full((LANES,), ts_mean, jnp.float32),
        jnp.full((LANES,), jnp.sqrt(ts_var + 1e-6), jnp.float32),
    ])

    mesh = plsc.VectorSubcoreMesh(core_axis_name="c", subcore_axis_name="s")
    repack = pl.kernel(
        _repack_body,
        out_type=jax.ShapeDtypeStruct((DIM * PACKW,), jnp.float32),
        mesh=mesh,
        scratch_types=[
            pltpu.VMEM((3, DIM, CHUNK), jnp.float32),
            pltpu.SemaphoreType.DMA,
            pltpu.SemaphoreType.DMA,
            pltpu.SemaphoreType.DMA,
            pltpu.SemaphoreType.DMA,
            pltpu.SemaphoreType.DMA,
            pltpu.SemaphoreType.DMA,
        ],
        compiler_params=pltpu.CompilerParams(use_tc_tiling_on_sc=True,
                                             needs_layout_passes=False),
    )
    flat_tab = repack(utab_t)

    lookup = pl.kernel(
        _lookup_body,
        out_type=jax.ShapeDtypeStruct((OUT_D, BATCH), jnp.float32),
        mesh=mesh,
        scratch_types=[
            pltpu.VMEM((B_PER_W,), jnp.int32),
            pltpu.VMEM((B_PER_W,), jnp.float32),
            pltpu.VMEM((NBND_PAD,), jnp.float32),
            pltpu.VMEM((DIM, B_PER_W), jnp.int32),
            pltpu.VMEM((OUT_D, B_PER_W), jnp.float32),
            pltpu.VMEM((1001, DIM), jnp.float32),
            pltpu.VMEM((NTAIL, DIM), jnp.float32),
            pltpu.VMEM((2 * LANES,), jnp.float32),
            pltpu.SemaphoreType.DMA,
        ],
        compiler_params=pltpu.CompilerParams(use_tc_tiling_on_sc=False,
                                             needs_layout_passes=False),
    )
    out_t = lookup(idx, timestamp, flat_tab, ts_table, bnd_pad, norm, tail)
    return out_t.T


# speedup vs baseline: 1.1190x; 1.1190x over previous
import jax
import jax.numpy as jnp
from jax import lax
from jax.experimental import pallas as pl
from jax.experimental.pallas import tpu as pltpu
from jax.experimental.pallas import tpu_sc as plsc

BATCH = 16384
DIM = 32
OUT_D = 2 * DIM + 1
VROWS = 1000001
NBLK = VROWS // 128
TAIL0 = NBLK * 128
NTAIL = VROWS - TAIL0
PACKW = TAIL0
NBND_PAD = 1024
LANES = 16

_NC, _NS = 2, 16
NW = _NC * _NS
B_PER_W = BATCH // NW
N_CHUNK = B_PER_W // LANES
BLK_PER_W = -(-NBLK // NW)
CHUNK = 5 * 128
N_CHUNKS_A = BLK_PER_W * 128 // CHUNK


def _repack_body(utab_t, flat_out, in_v, sem_i0, sem_i1, sem_i2,
                 sem_o0, sem_o1, sem_o2):
    wid = lax.axis_index("s") * _NC + lax.axis_index("c")
    c0w = jnp.minimum(wid * BLK_PER_W, NBLK - BLK_PER_W) * 128

    in_sems = [sem_i0, sem_i1, sem_i2]
    out_sems = [sem_o0, sem_o1, sem_o2]

    def issue_in(c0, buf):
        pltpu.async_copy(utab_t.at[pl.ds(0, DIM), pl.ds(c0, CHUNK)],
                         in_v.at[buf], in_sems[buf])

    def drain(buf, sem):
        pltpu.make_async_copy(utab_t.at[pl.ds(0, DIM), pl.ds(0, CHUNK)],
                              in_v.at[buf], sem).wait()

    issue_in(c0w, 0)

    def chunk_body(i, carry):
        c0 = c0w + i * CHUNK
        for cbuf in range(3):
            @pl.when(lax.rem(i, 3) == cbuf)
            def _():
                nbuf = (cbuf + 1) % 3

                @pl.when(i + 1 < N_CHUNKS_A)
                def _():
                    @pl.when(i >= 2)
                    def _():
                        drain(nbuf, out_sems[nbuf])
                    issue_in(c0 + CHUNK, nbuf)

                drain(cbuf, in_sems[cbuf])
                for j in range(DIM):
                    pltpu.async_copy(
                        in_v.at[cbuf, j],
                        flat_out.at[pl.ds(j * PACKW + c0, CHUNK)],
                        out_sems[cbuf])
        return carry

    lax.fori_loop(0, N_CHUNKS_A, chunk_body, 0)
    for cbuf in range(3):
        drain(cbuf, out_sems[cbuf])


def _lookup_body(uid_hbm, ts_hbm, flat_tab, ttab_hbm, bnd_hbm, norm_hbm,
                 tail_hbm, out_hbm,
                 idx_v, ts_v, bnd_v, cidx_v, cols_v, ttab_v, tail_v, norm_v,
                 sem_u, sem_s):
    wid = lax.axis_index("s") * _NC + lax.axis_index("c")
    base = wid * B_PER_W

    pltpu.sync_copy(uid_hbm.at[pl.ds(base, B_PER_W)], idx_v)
    staging = [
        pltpu.async_copy(ts_hbm.at[pl.ds(base, B_PER_W)], ts_v, sem_s),
        pltpu.async_copy(ttab_hbm, ttab_v, sem_s),
        pltpu.async_copy(tail_hbm, tail_v, sem_s),
        pltpu.async_copy(bnd_hbm, bnd_v, sem_s),
        pltpu.async_copy(norm_hbm, norm_v, sem_s),
    ]

    def cidx_body(j, off):
        for g in range(N_CHUNK):
            sl = pl.ds(g * LANES, LANES)
            iv = jnp.minimum(idx_v[sl], TAIL0 - 1)
            cidx_v[j, sl] = iv + off
        return off + PACKW

    lax.fori_loop(0, DIM, cidx_body, jnp.zeros((LANES,), jnp.int32))
    copies = [
        pltpu.async_copy(flat_tab.at[cidx_v.at[j]], cols_v.at[j], sem_u)
        for j in range(DIM)
    ]

    for c in staging:
        c.wait()
    mean = norm_v[pl.ds(0, LANES)]
    denom = norm_v[pl.ds(LANES, LANES)]

    def group_body(g, carry):
        sl = pl.ds(g * LANES, LANES)
        t = ts_v[sl]
        pos = jnp.zeros((LANES,), jnp.int32)
        step = NBND_PAD // 2
        while step >= 1:
            probe = pos + (step - 1)
            val = plsc.load_gather(bnd_v, [probe])
            pos = jnp.where(val <= t, pos + step, pos)
            step //= 2
        for j in range(DIM):
            tv = plsc.load_gather(ttab_v, [pos, jnp.full((LANES,), j,
                                                         jnp.int32)])
            cols_v[DIM + j, sl] = tv
        cols_v[2 * DIM, sl] = (t - mean) / denom
        return carry

    lax.fori_loop(0, N_CHUNK, group_body, 0)

    for c in copies:
        c.wait()

    def tail_body(g, carry):
        sl = pl.ds(g * LANES, LANES)
        iv = idx_v[sl]
        is_tail = iv >= TAIL0

        @pl.when(jnp.any(is_tail))
        def _():
            ti = jnp.maximum(iv - TAIL0, 0)
            for j in range(DIM):
                tv = plsc.load_gather(tail_v, [ti, jnp.full((LANES,), j,
                                                            jnp.int32)])
                cur = cols_v[j, sl]
                cols_v[j, sl] = jnp.where(is_tail, tv, cur)
        return carry

    lax.fori_loop(0, N_CHUNK, tail_body, 0)

    pltpu.sync_copy(cols_v, out_hbm.at[:, pl.ds(base, B_PER_W)])


@jax.jit
def kernel(user_id, timestamp, user_table, ts_table, boundaries, ts_mean,
           ts_var):
    idx = user_id.astype(jnp.int32)
    utab_t = user_table.T
    tail = user_table[TAIL0:]
    bnd_pad = jnp.full((NBND_PAD,), jnp.inf, jnp.float32).at[
        : boundaries.shape[0]].set(boundaries)
    norm = jnp.concatenate([
        jnp.full((LANES,), ts_mean, jnp.float32),
        jnp.full((LANES,), jnp.sqrt(ts_var + 1e-6), jnp.float32),
    ])

    mesh = plsc.VectorSubcoreMesh(core_axis_name="c", subcore_axis_name="s")
    repack = pl.kernel(
        _repack_body,
        out_type=jax.ShapeDtypeStruct((DIM * PACKW,), jnp.float32),
        mesh=mesh,
        scratch_types=[
            pltpu.VMEM((3, DIM, CHUNK), jnp.float32),
            pltpu.SemaphoreType.DMA,
            pltpu.SemaphoreType.DMA,
            pltpu.SemaphoreType.DMA,
            pltpu.SemaphoreType.DMA,
            pltpu.SemaphoreType.DMA,
            pltpu.SemaphoreType.DMA,
        ],
        compiler_params=pltpu.CompilerParams(use_tc_tiling_on_sc=True,
                                             needs_layout_passes=False),
    )
    flat_tab = repack(utab_t)

    lookup = pl.kernel(
        _lookup_body,
        out_type=jax.ShapeDtypeStruct((OUT_D, BATCH), jnp.float32),
        mesh=mesh,
        scratch_types=[
            pltpu.VMEM((B_PER_W,), jnp.int32),
            pltpu.VMEM((B_PER_W,), jnp.float32),
            pltpu.VMEM((NBND_PAD,), jnp.float32),
            pltpu.VMEM((DIM, B_PER_W), jnp.int32),
            pltpu.VMEM((OUT_D, B_PER_W), jnp.float32),
            pltpu.VMEM((1001, DIM), jnp.float32),
            pltpu.VMEM((NTAIL, DIM), jnp.float32),
            pltpu.VMEM((2 * LANES,), jnp.float32),
            pltpu.SemaphoreType.DMA,
            pltpu.SemaphoreType.DMA,
        ],
        compiler_params=pltpu.CompilerParams(use_tc_tiling_on_sc=False,
                                             needs_layout_passes=False),
    )
    out_t = lookup(idx, timestamp, flat_tab, ts_table, bnd_pad, norm, tail)
    return out_t.T
